# Initial kernel scaffold; baseline (speedup 1.0000x reference)
import jax
import jax.numpy as jnp
from jax.experimental import pallas as pl


def kernel(token_ids, embeddings):
    raise NotImplementedError("write your pallas kernel here")


# speedup vs baseline: 2.9636x; 2.9636x over previous
import functools

import jax
import jax.numpy as jnp
from jax import lax
from jax.experimental import pallas as pl
from jax.experimental.pallas import tpu as pltpu
from jax.experimental.pallas import tpu_sc as plsc

NUM_ROWS = 4096 * 50
DIM = 128
NC, NS = 2, 16
NW = NC * NS
B_PER_W = NUM_ROWS // NW
CHUNK = 128
N_CHUNKS = B_PER_W // CHUNK

_mesh = plsc.VectorSubcoreMesh(
    core_axis_name="c", subcore_axis_name="s", num_cores=NC, num_subcores=NS
)


@functools.partial(
    pl.kernel,
    out_type=jax.ShapeDtypeStruct((NUM_ROWS, DIM), jnp.float32),
    mesh=_mesh,
    scratch_types=[
        pltpu.VMEM((B_PER_W,), jnp.int32),
        pltpu.VMEM((CHUNK, DIM), jnp.float32),
        pltpu.SemaphoreType.DMA,
    ],
)
def _emb_lookup(idx_hbm, table_hbm, out_hbm, idx_v, rows_v, sem):
    wid = lax.axis_index("s") * NC + lax.axis_index("c")
    base = wid * B_PER_W
    pltpu.sync_copy(idx_hbm.at[pl.ds(base, B_PER_W)], idx_v)

    def body(g, carry):
        off = base + g * CHUNK
        ids = idx_v.at[pl.ds(g * CHUNK, CHUNK)]
        pltpu.async_copy(table_hbm.at[ids], rows_v, sem).wait()
        pltpu.sync_copy(rows_v, out_hbm.at[pl.ds(off, CHUNK)])
        return carry

    lax.fori_loop(0, N_CHUNKS, body, 0)


def kernel(token_ids, embeddings):
    flat_ids = token_ids.reshape(NUM_ROWS).astype(jnp.int32)
    out = _emb_lookup(flat_ids, embeddings)
    return out.reshape(*token_ids.shape, DIM)


# speedup vs baseline: 3.3258x; 1.1222x over previous
import functools

import jax
import jax.numpy as jnp
from jax import lax
from jax.experimental import pallas as pl
from jax.experimental.pallas import tpu as pltpu
from jax.experimental.pallas import tpu_sc as plsc

NUM_ROWS = 4096 * 50
DIM = 128
NC, NS = 2, 16
NW = NC * NS
B_PER_W = NUM_ROWS // NW
CHUNK = 800
N_CHUNKS = B_PER_W // CHUNK

_mesh = plsc.VectorSubcoreMesh(
    core_axis_name="c", subcore_axis_name="s", num_cores=NC, num_subcores=NS
)


@functools.partial(
    pl.kernel,
    out_type=jax.ShapeDtypeStruct((NUM_ROWS, DIM), jnp.float32),
    mesh=_mesh,
    scratch_types=[
        pltpu.VMEM((B_PER_W,), jnp.int32),
        pltpu.VMEM((CHUNK, DIM), jnp.float32),
        pltpu.SemaphoreType.DMA,
    ],
)
def _emb_lookup(idx_hbm, table_hbm, out_hbm, idx_v, rows_v, sem):
    wid = lax.axis_index("s") * NC + lax.axis_index("c")
    base = wid * B_PER_W
    pltpu.sync_copy(idx_hbm.at[pl.ds(base, B_PER_W)], idx_v)

    def body(g, carry):
        off = base + g * CHUNK
        ids = idx_v.at[pl.ds(g * CHUNK, CHUNK)]
        pltpu.async_copy(table_hbm.at[ids], rows_v, sem).wait()
        pltpu.sync_copy(rows_v, out_hbm.at[pl.ds(off, CHUNK)])
        return carry

    lax.fori_loop(0, N_CHUNKS, body, 0)


def kernel(token_ids, embeddings):
    flat_ids = token_ids.reshape(NUM_ROWS).astype(jnp.int32)
    out = _emb_lookup(flat_ids, embeddings)
    return out.reshape(*token_ids.shape, DIM)


# speedup vs baseline: 3.3376x; 1.0035x over previous
import functools

import jax
import jax.numpy as jnp
from jax import lax
from jax.experimental import pallas as pl
from jax.experimental.pallas import tpu as pltpu
from jax.experimental.pallas import tpu_sc as plsc

NUM_ROWS = 4096 * 50
DIM = 128
NC, NS = 2, 16
NW = NC * NS
B_PER_W = NUM_ROWS // NW
CHUNK = 320
N_CHUNKS = B_PER_W // CHUNK
NBUF = 3

_mesh = plsc.VectorSubcoreMesh(
    core_axis_name="c", subcore_axis_name="s", num_cores=NC, num_subcores=NS
)


@functools.partial(
    pl.kernel,
    out_type=jax.ShapeDtypeStruct((NUM_ROWS, DIM), jnp.float32),
    mesh=_mesh,
    scratch_types=[
        pltpu.VMEM((B_PER_W,), jnp.int32),
        pltpu.VMEM((NBUF, CHUNK, DIM), jnp.float32),
        [pltpu.SemaphoreType.DMA] * NBUF,
        [pltpu.SemaphoreType.DMA] * NBUF,
    ],
)
def _emb_lookup(idx_hbm, table_hbm, out_hbm, idx_v, rows_v, gsem, wsem):
    wid = lax.axis_index("s") * NC + lax.axis_index("c")
    base = wid * B_PER_W
    pltpu.sync_copy(idx_hbm.at[pl.ds(base, B_PER_W)], idx_v)

    def ids_of(c):
        return idx_v.at[pl.ds(c * CHUNK, CHUNK)]

    def out_of(c):
        return out_hbm.at[pl.ds(base + c * CHUNK, CHUNK)]

    for b in range(NBUF):
        pltpu.async_copy(table_hbm.at[ids_of(b)], rows_v.at[b], gsem[b])

    for c in range(N_CHUNKS):
        b = c % NBUF
        pltpu.make_async_copy(table_hbm.at[ids_of(c)], rows_v.at[b], gsem[b]).wait()
        pltpu.async_copy(rows_v.at[b], out_of(c), wsem[b])
        nxt = c + NBUF
        if nxt < N_CHUNKS:
            pltpu.make_async_copy(rows_v.at[b], out_of(c), wsem[b]).wait()
            pltpu.async_copy(table_hbm.at[ids_of(nxt)], rows_v.at[b], gsem[b])

    for c in range(N_CHUNKS - NBUF, N_CHUNKS):
        b = c % NBUF
        pltpu.make_async_copy(rows_v.at[b], out_of(c), wsem[b]).wait()


def kernel(token_ids, embeddings):
    flat_ids = token_ids.reshape(NUM_ROWS).astype(jnp.int32)
    out = _emb_lookup(flat_ids, embeddings)
    return out.reshape(*token_ids.shape, DIM)


# speedup vs baseline: 5.8094x; 1.7406x over previous
import functools

import jax
import jax.numpy as jnp
from jax import lax
from jax.experimental import pallas as pl
from jax.experimental.pallas import tpu as pltpu
from jax.experimental.pallas import tpu_sc as plsc

NUM_SAMPLES = 4096
SEQ = 50
NUM_ROWS = NUM_SAMPLES * SEQ
DIM = 128
NC, NS = 2, 16
NW = NC * NS
S_PER_W = NUM_SAMPLES // NW
B_PER_W = S_PER_W * SEQ
S_CHUNK = 8
CHUNK = S_CHUNK * SEQ
N_CHUNKS = S_PER_W // S_CHUNK
NBUF = 2

_mesh = plsc.VectorSubcoreMesh(
    core_axis_name="c", subcore_axis_name="s", num_cores=NC, num_subcores=NS
)


@functools.partial(
    pl.kernel,
    out_type=jax.ShapeDtypeStruct((NUM_SAMPLES, SEQ, DIM), jnp.float32),
    mesh=_mesh,
    compiler_params=pltpu.CompilerParams(use_tc_tiling_on_sc=True),
    scratch_types=[
        pltpu.VMEM((B_PER_W,), jnp.int32),
        pltpu.VMEM((NBUF, CHUNK, DIM), jnp.float32),
        [pltpu.SemaphoreType.DMA] * NBUF,
        [pltpu.SemaphoreType.DMA] * NBUF,
    ],
)
def _emb_lookup(idx_hbm, table_hbm, out_hbm, idx_v, rows_v, gsem, wsem):
    wid = lax.axis_index("s") * NC + lax.axis_index("c")
    base = wid * B_PER_W
    s_base = wid * S_PER_W
    pltpu.sync_copy(idx_hbm.at[pl.ds(base, B_PER_W)], idx_v)

    def ids_of(c):
        return idx_v.at[pl.ds(c * CHUNK, CHUNK)]

    def writes_of(c, b):
        i0 = s_base + c * S_CHUNK
        return [
            (rows_v.at[b, pl.ds(s * SEQ, SEQ)], out_hbm.at[i0 + s])
            for s in range(S_CHUNK)
        ]

    for b in range(NBUF):
        pltpu.async_copy(table_hbm.at[ids_of(b)], rows_v.at[b], gsem[b])

    for c in range(N_CHUNKS):
        b = c % NBUF
        pltpu.make_async_copy(table_hbm.at[ids_of(c)], rows_v.at[b], gsem[b]).wait()
        for src, dst in writes_of(c, b):
            pltpu.async_copy(src, dst, wsem[b])
        nxt = c + NBUF
        if nxt < N_CHUNKS:
            for src, dst in writes_of(c, b):
                pltpu.make_async_copy(src, dst, wsem[b]).wait()
            pltpu.async_copy(table_hbm.at[ids_of(nxt)], rows_v.at[b], gsem[b])

    for c in range(N_CHUNKS - NBUF, N_CHUNKS):
        b = c % NBUF
        for src, dst in writes_of(c, b):
            pltpu.make_async_copy(src, dst, wsem[b]).wait()


def kernel(token_ids, embeddings):
    flat_ids = token_ids.reshape(NUM_ROWS).astype(jnp.int32)
    return _emb_lookup(flat_ids, embeddings)
